# baseline (speedup 1.0000x reference)
import jax
import jax.numpy as jnp
from jax.experimental import pallas as pl

_N_USERS = 8192
_N_ITEMS = 4096
_D = 64
_IMG_DIM = 4096
_TXT_DIM = 384
_EPS_NOISE = 0.2
_CAT = 0.55

_F32 = jnp.float32
_BF16 = jnp.bfloat16


def _row_l2norm(x, eps=1e-12):
    n = jnp.sqrt(jnp.sum(x * x, axis=1, keepdims=True))
    return x / jnp.clip(n, eps, None)


def _stage_a(img_ref, txt_ref, emb_ref, wimg_ref, bimg_ref, wtxt_ref, btxt_ref,
             out_ref):
    imf = jnp.dot(img_ref[...].astype(_BF16), wimg_ref[...],
                  preferred_element_type=_F32)
    txf = jnp.dot(txt_ref[...].astype(_BF16), wtxt_ref[...],
                  preferred_element_type=_F32)
    out_ref[...] = jnp.concatenate(
        [imf + bimg_ref[...], txf + btxt_ref[...], emb_ref[...]],
        axis=1).astype(_BF16)


def _stage_prop_noise(g_ref, rhs_ref, noise_ref,
                      img_ref, txt_ref, emb_ref, outm_ref, embm_ref):
    prod = jnp.dot(g_ref[...].astype(_BF16), rhs_ref[...],
                   preferred_element_type=_F32)
    raw = prod[:, 2 * _D:]
    noised = raw + jnp.sign(raw) * _row_l2norm(noise_ref[...]) * _EPS_NOISE
    img_ref[...] = prod[:, :_D]
    txt_ref[...] = prod[:, _D:2 * _D]
    emb_ref[...] = noised
    outm_ref[...] = prod.astype(_BF16)
    embm_ref[...] = noised.astype(_BF16)


def _stage_user_final(ui_ref, i1n_ref, uemb_ref, imgu_ref, txtu_ref, u1n_ref,
                      uout_ref, ucl_ref, ug2_ref):
    ug2 = jnp.dot(ui_ref[...].astype(_BF16), i1n_ref[...],
                  preferred_element_type=_F32)
    mix = _CAT * _row_l2norm(imgu_ref[...]) + _CAT * _row_l2norm(txtu_ref[...])
    mean = (uemb_ref[...] + u1n_ref[...] + ug2) * (1.0 / 3.0)
    uout_ref[...] = mean + mix
    ucl_ref[...] = ug2 + mix
    ug2_ref[...] = ug2.astype(_BF16)


def _stage_item_final(iu_ref, ug2_ref, iemb_ref, imgi_ref, txti_ref, i1n_ref,
                      iout_ref, icl_ref):
    ig2 = jnp.dot(iu_ref[...].astype(_BF16), ug2_ref[...],
                  preferred_element_type=_F32)
    mix = _CAT * _row_l2norm(imgi_ref[...]) + _CAT * _row_l2norm(txti_ref[...])
    mean = (iemb_ref[...] + i1n_ref[...] + ig2) * (1.0 / 3.0)
    iout_ref[...] = mean + mix
    icl_ref[...] = ig2 + mix


def kernel(ui_graph, iu_graph, image_feats, text_feats, user_emb, item_emb,
           W_img, b_img, W_txt, b_txt):
    ND = 3 * _D

    nkey = jax.random.key(42)
    u_noise = jax.random.uniform(jax.random.fold_in(nkey, 0), (_N_USERS, _D),
                                 dtype=_F32)
    i_noise = jax.random.uniform(jax.random.fold_in(nkey, 1), (_N_ITEMS, _D),
                                 dtype=_F32)

    wimg_t = W_img.T.astype(_BF16)
    wtxt_t = W_txt.T.astype(_BF16)
    bimg = b_img.reshape(1, _D)
    btxt = b_txt.reshape(1, _D)

    def _row(bm):
        return lambda i: (i, 0)

    def _full():
        return lambda i: (0, 0)

    bm_a = 512
    r0 = pl.pallas_call(
        _stage_a,
        grid=(_N_ITEMS // bm_a,),
        in_specs=[
            pl.BlockSpec((bm_a, _IMG_DIM), _row(bm_a)),
            pl.BlockSpec((bm_a, _TXT_DIM), _row(bm_a)),
            pl.BlockSpec((bm_a, _D), _row(bm_a)),
            pl.BlockSpec((_IMG_DIM, _D), _full()),
            pl.BlockSpec((1, _D), _full()),
            pl.BlockSpec((_TXT_DIM, _D), _full()),
            pl.BlockSpec((1, _D), _full()),
        ],
        out_specs=pl.BlockSpec((bm_a, ND), _row(bm_a)),
        out_shape=jax.ShapeDtypeStruct((_N_ITEMS, ND), _BF16),
    )(image_feats, text_feats, item_emb, wimg_t, bimg, wtxt_t, btxt)

    bm_b = 256
    image_user_feats, text_user_feats, u1n, u1m, _u1nm = pl.pallas_call(
        _stage_prop_noise,
        grid=(_N_USERS // bm_b,),
        in_specs=[
            pl.BlockSpec((bm_b, _N_ITEMS), _row(bm_b)),
            pl.BlockSpec((_N_ITEMS, ND), _full()),
            pl.BlockSpec((bm_b, _D), _row(bm_b)),
        ],
        out_specs=[
            pl.BlockSpec((bm_b, _D), _row(bm_b)),
            pl.BlockSpec((bm_b, _D), _row(bm_b)),
            pl.BlockSpec((bm_b, _D), _row(bm_b)),
            pl.BlockSpec((bm_b, ND), _row(bm_b)),
            pl.BlockSpec((bm_b, _D), _row(bm_b)),
        ],
        out_shape=[
            jax.ShapeDtypeStruct((_N_USERS, _D), _F32),
            jax.ShapeDtypeStruct((_N_USERS, _D), _F32),
            jax.ShapeDtypeStruct((_N_USERS, _D), _F32),
            jax.ShapeDtypeStruct((_N_USERS, ND), _BF16),
            jax.ShapeDtypeStruct((_N_USERS, _D), _BF16),
        ],
    )(ui_graph, r0, u_noise)

    bm_c = 128
    image_item_feats, text_item_feats, i1n, _i1m, i1nm = pl.pallas_call(
        _stage_prop_noise,
        grid=(_N_ITEMS // bm_c,),
        in_specs=[
            pl.BlockSpec((bm_c, _N_USERS), _row(bm_c)),
            pl.BlockSpec((_N_USERS, ND), _full()),
            pl.BlockSpec((bm_c, _D), _row(bm_c)),
        ],
        out_specs=[
            pl.BlockSpec((bm_c, _D), _row(bm_c)),
            pl.BlockSpec((bm_c, _D), _row(bm_c)),
            pl.BlockSpec((bm_c, _D), _row(bm_c)),
            pl.BlockSpec((bm_c, ND), _row(bm_c)),
            pl.BlockSpec((bm_c, _D), _row(bm_c)),
        ],
        out_shape=[
            jax.ShapeDtypeStruct((_N_ITEMS, _D), _F32),
            jax.ShapeDtypeStruct((_N_ITEMS, _D), _F32),
            jax.ShapeDtypeStruct((_N_ITEMS, _D), _F32),
            jax.ShapeDtypeStruct((_N_ITEMS, ND), _BF16),
            jax.ShapeDtypeStruct((_N_ITEMS, _D), _BF16),
        ],
    )(iu_graph, u1m, i_noise)

    bm_d = 256
    u_out, u_cl_out, u_g2 = pl.pallas_call(
        _stage_user_final,
        grid=(_N_USERS // bm_d,),
        in_specs=[
            pl.BlockSpec((bm_d, _N_ITEMS), _row(bm_d)),
            pl.BlockSpec((_N_ITEMS, _D), _full()),
            pl.BlockSpec((bm_d, _D), _row(bm_d)),
            pl.BlockSpec((bm_d, _D), _row(bm_d)),
            pl.BlockSpec((bm_d, _D), _row(bm_d)),
            pl.BlockSpec((bm_d, _D), _row(bm_d)),
        ],
        out_specs=[
            pl.BlockSpec((bm_d, _D), _row(bm_d)),
            pl.BlockSpec((bm_d, _D), _row(bm_d)),
            pl.BlockSpec((bm_d, _D), _row(bm_d)),
        ],
        out_shape=[
            jax.ShapeDtypeStruct((_N_USERS, _D), _F32),
            jax.ShapeDtypeStruct((_N_USERS, _D), _F32),
            jax.ShapeDtypeStruct((_N_USERS, _D), _BF16),
        ],
    )(ui_graph, i1nm, user_emb, image_user_feats, text_user_feats, u1n)

    bm_e = 128
    i_out, i_cl_out = pl.pallas_call(
        _stage_item_final,
        grid=(_N_ITEMS // bm_e,),
        in_specs=[
            pl.BlockSpec((bm_e, _N_USERS), _row(bm_e)),
            pl.BlockSpec((_N_USERS, _D), _full()),
            pl.BlockSpec((bm_e, _D), _row(bm_e)),
            pl.BlockSpec((bm_e, _D), _row(bm_e)),
            pl.BlockSpec((bm_e, _D), _row(bm_e)),
            pl.BlockSpec((bm_e, _D), _row(bm_e)),
        ],
        out_specs=[
            pl.BlockSpec((bm_e, _D), _row(bm_e)),
            pl.BlockSpec((bm_e, _D), _row(bm_e)),
        ],
        out_shape=[
            jax.ShapeDtypeStruct((_N_ITEMS, _D), _F32),
            jax.ShapeDtypeStruct((_N_ITEMS, _D), _F32),
        ],
    )(iu_graph, u_g2, item_emb, image_item_feats, text_item_feats, i1n)

    prompt_user = jnp.zeros((_N_USERS, _D), dtype=_F32)
    prompt_item = jnp.zeros((_N_ITEMS, _D), dtype=_F32)
    gcl_loss = jnp.float32(0.0)

    return (u_out, i_out, image_item_feats, text_item_feats,
            image_user_feats, text_user_feats, u_out, i_out,
            prompt_user, prompt_item, gcl_loss)


# speedup vs baseline: 1.1824x; 1.1824x over previous
import jax
import jax.numpy as jnp
from jax.experimental import pallas as pl
from jax.experimental.pallas import tpu as pltpu

_N_USERS = 8192
_N_ITEMS = 4096
_D = 64
_ND = 3 * _D
_IMG_DIM = 4096
_TXT_DIM = 384
_EPS_NOISE = 0.2
_CAT = 0.55

_F32 = jnp.float32
_BF16 = jnp.bfloat16

_BM_U = 512
_BM_I = 256
_BM_A = 512


def _row_l2norm(x, eps=1e-12):
    n = jnp.sqrt(jnp.sum(x * x, axis=1, keepdims=True))
    return x / jnp.clip(n, eps, None)


def _row(i):
    return (i, 0)


def _mono(ui, iu, img, txt, uemb, iemb, wimg, bimg, wtxt, btxt, unoise,
          inoise,
          uout_o, iout_o, imgi_o, txti_o, imgu_o, txtu_o, ucl_o, icl_o,
          r0_v, u1m_v, u1n_v, imgu_v, txtu_v, i1nm_v, i1n_v, imgi_v, txti_v,
          ug2_v):
    def a_body(img_b, txt_b, iemb_b, r0_b):
        imf = jnp.dot(img_b[...].astype(_BF16), wimg[...],
                      preferred_element_type=_F32)
        txf = jnp.dot(txt_b[...].astype(_BF16), wtxt[...],
                      preferred_element_type=_F32)
        r0_b[...] = jnp.concatenate(
            [imf + bimg[...], txf + btxt[...], iemb_b[...]],
            axis=1).astype(_BF16)

    pltpu.emit_pipeline(
        a_body,
        grid=(_N_ITEMS // _BM_A,),
        in_specs=[
            pl.BlockSpec((_BM_A, _IMG_DIM), _row),
            pl.BlockSpec((_BM_A, _TXT_DIM), _row),
            pl.BlockSpec((_BM_A, _D), _row),
        ],
        out_specs=[pl.BlockSpec((_BM_A, _ND), _row)],
    )(img, txt, iemb, r0_v)

    def b_body(ui_b, unoise_b, imgu_ob, txtu_ob, imgu_vb, txtu_vb, u1n_vb,
               u1m_vb):
        prod = jnp.dot(ui_b[...].astype(_BF16), r0_v[...],
                       preferred_element_type=_F32)
        raw = prod[:, 2 * _D:]
        noised = raw + jnp.sign(raw) * _row_l2norm(unoise_b[...]) * _EPS_NOISE
        imgu = prod[:, :_D]
        txtu = prod[:, _D:2 * _D]
        imgu_ob[...] = imgu
        txtu_ob[...] = txtu
        imgu_vb[...] = imgu
        txtu_vb[...] = txtu
        u1n_vb[...] = noised
        u1m_vb[...] = prod.astype(_BF16)

    pltpu.emit_pipeline(
        b_body,
        grid=(_N_USERS // _BM_U,),
        in_specs=[
            pl.BlockSpec((_BM_U, _N_ITEMS), _row),
            pl.BlockSpec((_BM_U, _D), _row),
        ],
        out_specs=[
            pl.BlockSpec((_BM_U, _D), _row),
            pl.BlockSpec((_BM_U, _D), _row),
            pl.BlockSpec((_BM_U, _D), _row),
            pl.BlockSpec((_BM_U, _D), _row),
            pl.BlockSpec((_BM_U, _D), _row),
            pl.BlockSpec((_BM_U, _ND), _row),
        ],
    )(ui, unoise, imgu_o, txtu_o, imgu_v, txtu_v, u1n_v, u1m_v)

    def c_body(iu_b, inoise_b, imgi_ob, txti_ob, imgi_vb, txti_vb, i1n_vb,
               i1nm_vb):
        prod = jnp.dot(iu_b[...].astype(_BF16), u1m_v[...],
                       preferred_element_type=_F32)
        raw = prod[:, 2 * _D:]
        noised = raw + jnp.sign(raw) * _row_l2norm(inoise_b[...]) * _EPS_NOISE
        imgi = prod[:, :_D]
        txti = prod[:, _D:2 * _D]
        imgi_ob[...] = imgi
        txti_ob[...] = txti
        imgi_vb[...] = imgi
        txti_vb[...] = txti
        i1n_vb[...] = noised
        i1nm_vb[...] = noised.astype(_BF16)

    pltpu.emit_pipeline(
        c_body,
        grid=(_N_ITEMS // _BM_I,),
        in_specs=[
            pl.BlockSpec((_BM_I, _N_USERS), _row),
            pl.BlockSpec((_BM_I, _D), _row),
        ],
        out_specs=[pl.BlockSpec((_BM_I, _D), _row)] * 6,
    )(iu, inoise, imgi_o, txti_o, imgi_v, txti_v, i1n_v, i1nm_v)

    def d_body(ui_b, uemb_b, imgu_vb, txtu_vb, u1n_vb, uout_ob, ucl_ob,
               ug2_vb):
        ug2 = jnp.dot(ui_b[...].astype(_BF16), i1nm_v[...],
                      preferred_element_type=_F32)
        mix = (_CAT * _row_l2norm(imgu_vb[...])
               + _CAT * _row_l2norm(txtu_vb[...]))
        mean = (uemb_b[...] + u1n_vb[...] + ug2) * (1.0 / 3.0)
        uout_ob[...] = mean + mix
        ucl_ob[...] = ug2 + mix
        ug2_vb[...] = ug2.astype(_BF16)

    pltpu.emit_pipeline(
        d_body,
        grid=(_N_USERS // _BM_U,),
        in_specs=[
            pl.BlockSpec((_BM_U, _N_ITEMS), _row),
            pl.BlockSpec((_BM_U, _D), _row),
            pl.BlockSpec((_BM_U, _D), _row),
            pl.BlockSpec((_BM_U, _D), _row),
            pl.BlockSpec((_BM_U, _D), _row),
        ],
        out_specs=[pl.BlockSpec((_BM_U, _D), _row)] * 3,
    )(ui, uemb, imgu_v, txtu_v, u1n_v, uout_o, ucl_o, ug2_v)

    def e_body(iu_b, iemb_b, imgi_vb, txti_vb, i1n_vb, iout_ob, icl_ob):
        ig2 = jnp.dot(iu_b[...].astype(_BF16), ug2_v[...],
                      preferred_element_type=_F32)
        mix = (_CAT * _row_l2norm(imgi_vb[...])
               + _CAT * _row_l2norm(txti_vb[...]))
        mean = (iemb_b[...] + i1n_vb[...] + ig2) * (1.0 / 3.0)
        iout_ob[...] = mean + mix
        icl_ob[...] = ig2 + mix

    pltpu.emit_pipeline(
        e_body,
        grid=(_N_ITEMS // _BM_I,),
        in_specs=[
            pl.BlockSpec((_BM_I, _N_USERS), _row),
            pl.BlockSpec((_BM_I, _D), _row),
            pl.BlockSpec((_BM_I, _D), _row),
            pl.BlockSpec((_BM_I, _D), _row),
            pl.BlockSpec((_BM_I, _D), _row),
        ],
        out_specs=[pl.BlockSpec((_BM_I, _D), _row)] * 2,
    )(iu, iemb, imgi_v, txti_v, i1n_v, iout_o, icl_o)


def kernel(ui_graph, iu_graph, image_feats, text_feats, user_emb, item_emb,
           W_img, b_img, W_txt, b_txt):
    nkey = jax.random.key(42)
    u_noise = jax.random.uniform(jax.random.fold_in(nkey, 0), (_N_USERS, _D),
                                 dtype=_F32)
    i_noise = jax.random.uniform(jax.random.fold_in(nkey, 1), (_N_ITEMS, _D),
                                 dtype=_F32)

    wimg_t = W_img.T.astype(_BF16)
    wtxt_t = W_txt.T.astype(_BF16)
    bimg = b_img.reshape(1, _D)
    btxt = b_txt.reshape(1, _D)

    any_spec = pl.BlockSpec(memory_space=pl.ANY)
    vmem_spec = pl.BlockSpec(memory_space=pltpu.MemorySpace.VMEM)

    (u_out, i_out, image_item_feats, text_item_feats, image_user_feats,
     text_user_feats, u_cl_out, i_cl_out) = pl.pallas_call(
        _mono,
        in_specs=[
            any_spec, any_spec, any_spec, any_spec, any_spec, any_spec,
            vmem_spec, vmem_spec, vmem_spec, vmem_spec,
            any_spec, any_spec,
        ],
        out_specs=[any_spec] * 8,
        out_shape=[
            jax.ShapeDtypeStruct((_N_USERS, _D), _F32),
            jax.ShapeDtypeStruct((_N_ITEMS, _D), _F32),
            jax.ShapeDtypeStruct((_N_ITEMS, _D), _F32),
            jax.ShapeDtypeStruct((_N_ITEMS, _D), _F32),
            jax.ShapeDtypeStruct((_N_USERS, _D), _F32),
            jax.ShapeDtypeStruct((_N_USERS, _D), _F32),
            jax.ShapeDtypeStruct((_N_USERS, _D), _F32),
            jax.ShapeDtypeStruct((_N_ITEMS, _D), _F32),
        ],
        scratch_shapes=[
            pltpu.VMEM((_N_ITEMS, _ND), _BF16),
            pltpu.VMEM((_N_USERS, _ND), _BF16),
            pltpu.VMEM((_N_USERS, _D), _F32),
            pltpu.VMEM((_N_USERS, _D), _F32),
            pltpu.VMEM((_N_USERS, _D), _F32),
            pltpu.VMEM((_N_ITEMS, _D), _BF16),
            pltpu.VMEM((_N_ITEMS, _D), _F32),
            pltpu.VMEM((_N_ITEMS, _D), _F32),
            pltpu.VMEM((_N_ITEMS, _D), _F32),
            pltpu.VMEM((_N_USERS, _D), _BF16),
        ],
    )(ui_graph, iu_graph, image_feats, text_feats, user_emb, item_emb,
      wimg_t, bimg, wtxt_t, btxt, u_noise, i_noise)

    prompt_user = jnp.zeros((_N_USERS, _D), dtype=_F32)
    prompt_item = jnp.zeros((_N_ITEMS, _D), dtype=_F32)
    gcl_loss = jnp.float32(0.0)

    return (u_out, i_out, image_item_feats, text_item_feats,
            image_user_feats, text_user_feats, u_out, i_out,
            prompt_user, prompt_item, gcl_loss)
